# baseline (speedup 1.0000x reference)
import functools

import jax
import jax.numpy as jnp
from jax import lax
from jax.experimental import pallas as pl
from jax.experimental.pallas import tpu as pltpu
from jax.experimental.pallas import tpu_sc as plsc

_NC, _NS = 2, 16
_NW = _NC * _NS
_CH = 256
_TC_BLK = 4096


@functools.lru_cache(maxsize=None)
def _make_transpose(n_rows, d):
    grid = (n_rows + _TC_BLK - 1) // _TC_BLK

    def body(x_ref, o_ref):
        o_ref[...] = x_ref[...].T

    return pl.pallas_call(
        body,
        grid=(grid,),
        in_specs=[pl.BlockSpec((d, _TC_BLK), lambda j: (0, j))],
        out_specs=pl.BlockSpec((_TC_BLK, d), lambda j: (j, 0)),
        out_shape=jax.ShapeDtypeStruct((n_rows, d), jnp.float32),
    )


@functools.lru_cache(maxsize=None)
def _make_gather(t_total, d):
    n_units = t_total // _CH
    upw = n_units // _NW
    assert upw * _NW * _CH == t_total

    mesh = plsc.VectorSubcoreMesh(core_axis_name="c", subcore_axis_name="s")

    @functools.partial(
        pl.kernel,
        mesh=mesh,
        compiler_params=pltpu.CompilerParams(use_tc_tiling_on_sc=False),
        out_type=jax.ShapeDtypeStruct((t_total, d), jnp.float32),
        scratch_types=[
            pltpu.VMEM((2, _CH), jnp.int32),
            pltpu.VMEM((2, _CH, d), jnp.float32),
            pltpu.SemaphoreType.DMA,
            pltpu.SemaphoreType.DMA,
            pltpu.SemaphoreType.DMA,
            pltpu.SemaphoreType.DMA,
        ],
    )
    def emb(table_hbm, idx_hbm, out_hbm, idx_v, rows_v, g0, g1, w0, w1):
        wid = lax.axis_index("s") * _NC + lax.axis_index("c")
        u0 = wid * upw
        gsem = (g0, g1)
        wsem = (w0, w1)
        gath = [None, None]
        wrb = [None, None]
        for i in range(upw):
            cur = i % 2
            u = u0 + i
            if wrb[cur] is not None:
                wrb[cur].wait()
            pltpu.sync_copy(idx_hbm.at[pl.ds(u * _CH, _CH)],
                            idx_v.at[cur])
            gath[cur] = pltpu.async_copy(
                table_hbm.at[idx_v.at[cur]], rows_v.at[cur], gsem[cur])
            if i > 0:
                prv = 1 - cur
                gath[prv].wait()
                wrb[prv] = pltpu.async_copy(
                    rows_v.at[prv],
                    out_hbm.at[pl.ds((u - 1) * _CH, _CH)],
                    wsem[prv])
        last = (upw - 1) % 2
        ul = u0 + upw - 1
        gath[last].wait()
        pltpu.sync_copy(rows_v.at[last],
                        out_hbm.at[pl.ds(ul * _CH, _CH)])
        if wrb[1 - last] is not None:
            wrb[1 - last].wait()

    return emb


def kernel(token_ids, weight):
    b, l = token_ids.shape
    n, d = weight.shape
    idx_flat = token_ids.reshape(-1).astype(jnp.int32)
    w_rows = _make_transpose(n, d)(weight.T)
    out = _make_gather(b * l, d)(w_rows, idx_flat)
    return out.reshape(b, l, d)


# speedup vs baseline: 1.1579x; 1.1579x over previous
import functools

import jax
import jax.numpy as jnp
from jax import lax
from jax.experimental import pallas as pl
from jax.experimental.pallas import tpu as pltpu
from jax.experimental.pallas import tpu_sc as plsc

_NC, _NS = 2, 16
_NW = _NC * _NS
_CH = 256
_TC_BLK = 4096


@functools.lru_cache(maxsize=None)
def _make_transpose(n_rows, d):
    grid = (n_rows + _TC_BLK - 1) // _TC_BLK

    def body(x_ref, o_ref):
        o_ref[...] = x_ref[...].T

    return pl.pallas_call(
        body,
        grid=(grid,),
        in_specs=[pl.BlockSpec((d, _TC_BLK), lambda j: (0, j))],
        out_specs=pl.BlockSpec((_TC_BLK, d), lambda j: (j, 0)),
        out_shape=jax.ShapeDtypeStruct((n_rows, d), jnp.float32),
    )


@functools.lru_cache(maxsize=None)
def _make_gather(t_total, d):
    n_units = t_total // _CH
    upw = n_units // _NW
    assert upw * _NW * _CH == t_total

    mesh = plsc.VectorSubcoreMesh(core_axis_name="c", subcore_axis_name="s")

    @functools.partial(
        pl.kernel,
        mesh=mesh,
        compiler_params=pltpu.CompilerParams(use_tc_tiling_on_sc=False),
        out_type=jax.ShapeDtypeStruct((t_total, d), jnp.float32),
        scratch_types=[
            pltpu.VMEM((2, _CH), jnp.int32),
            pltpu.VMEM((2, _CH, d), jnp.float32),
            pltpu.SemaphoreType.DMA,
            pltpu.SemaphoreType.DMA,
            pltpu.SemaphoreType.DMA,
            pltpu.SemaphoreType.DMA,
        ],
    )
    def emb(table_hbm, idx_hbm, out_hbm, idx_v, rows_v, g0, g1, w0, w1):
        wid = lax.axis_index("s") * _NC + lax.axis_index("c")
        u0 = wid * upw
        gsem = (g0, g1)
        wsem = (w0, w1)
        gath = [None, None]
        wrb = [None, None]
        for i in range(upw):
            cur = i % 2
            u = u0 + i
            if wrb[cur] is not None:
                wrb[cur].wait()
            pltpu.sync_copy(idx_hbm.at[pl.ds(u * _CH, _CH)],
                            idx_v.at[cur])
            gath[cur] = pltpu.async_copy(
                table_hbm.at[idx_v.at[cur]], rows_v.at[cur], gsem[cur])
            if i > 0:
                prv = 1 - cur
                gath[prv].wait()
                wrb[prv] = pltpu.async_copy(
                    rows_v.at[prv],
                    out_hbm.at[pl.ds((u - 1) * _CH, _CH)],
                    wsem[prv])
        last = (upw - 1) % 2
        ul = u0 + upw - 1
        gath[last].wait()
        pltpu.sync_copy(rows_v.at[last],
                        out_hbm.at[pl.ds(ul * _CH, _CH)])
        if wrb[1 - last] is not None:
            wrb[1 - last].wait()

    return emb


def kernel(token_ids, weight):
    b, l = token_ids.shape
    n, d = weight.shape
    idx_flat = token_ids.reshape(-1).astype(jnp.int32)
    out = _make_gather(b * l, d)(weight, idx_flat)
    return out.reshape(b, l, d)


# speedup vs baseline: 1.2271x; 1.0598x over previous
import functools

import jax
import jax.numpy as jnp
from jax import lax
from jax.experimental import pallas as pl
from jax.experimental.pallas import tpu as pltpu
from jax.experimental.pallas import tpu_sc as plsc

_NC, _NS = 2, 16
_NW = _NC * _NS
_CH = 256
_TC_BLK = 4096


@functools.lru_cache(maxsize=None)
def _make_transpose(n_rows, d):
    grid = (n_rows + _TC_BLK - 1) // _TC_BLK

    def body(x_ref, o_ref):
        o_ref[...] = x_ref[...].T

    return pl.pallas_call(
        body,
        grid=(grid,),
        in_specs=[pl.BlockSpec((d, _TC_BLK), lambda j: (0, j))],
        out_specs=pl.BlockSpec((_TC_BLK, d), lambda j: (j, 0)),
        out_shape=jax.ShapeDtypeStruct((n_rows, d), jnp.float32),
    )


@functools.lru_cache(maxsize=None)
def _make_gather(t_total, d):
    n_units = t_total // _CH
    upw = n_units // _NW
    assert upw * _NW * _CH == t_total

    mesh = plsc.VectorSubcoreMesh(core_axis_name="c", subcore_axis_name="s")

    @functools.partial(
        pl.kernel,
        mesh=mesh,
        compiler_params=pltpu.CompilerParams(use_tc_tiling_on_sc=False),
        out_type=jax.ShapeDtypeStruct((t_total, d), jnp.float32),
        scratch_types=[
            pltpu.VMEM((2, _CH), jnp.int32),
            pltpu.VMEM((2, _CH, 2 * d), jnp.float32),
            pltpu.SemaphoreType.DMA,
            pltpu.SemaphoreType.DMA,
            pltpu.SemaphoreType.DMA,
            pltpu.SemaphoreType.DMA,
        ],
    )
    def emb(table_hbm, idx_hbm, out_hbm, idx_v, rows_v, g0, g1, w0, w1):
        wid = lax.axis_index("s") * _NC + lax.axis_index("c")
        u0 = wid * upw
        gsem = (g0, g1)
        wsem = (w0, w1)
        gath = [None, None]
        wrb = [None, None]
        for i in range(upw):
            cur = i % 2
            u = u0 + i
            if wrb[cur] is not None:
                wrb[cur].wait()
            pltpu.sync_copy(idx_hbm.at[pl.ds(u * _CH, _CH)],
                            idx_v.at[cur])
            gath[cur] = pltpu.async_copy(
                table_hbm.at[idx_v.at[cur]], rows_v.at[cur], gsem[cur])
            if i > 0:
                prv = 1 - cur
                gath[prv].wait()
                wrb[prv] = pltpu.async_copy(
                    rows_v.at[prv, :, pl.ds(0, d)],
                    out_hbm.at[pl.ds((u - 1) * _CH, _CH)],
                    wsem[prv])
        last = (upw - 1) % 2
        ul = u0 + upw - 1
        gath[last].wait()
        pltpu.sync_copy(rows_v.at[last, :, pl.ds(0, d)],
                        out_hbm.at[pl.ds(ul * _CH, _CH)])
        if wrb[1 - last] is not None:
            wrb[1 - last].wait()

    return emb


def kernel(token_ids, weight):
    b, l = token_ids.shape
    n, d = weight.shape
    idx_flat = token_ids.reshape(-1).astype(jnp.int32)
    w_pad = jnp.pad(weight, ((0, 0), (0, d)))
    out = _make_gather(b * l, d)(w_pad, idx_flat)
    return out.reshape(b, l, d)


# speedup vs baseline: 1.2385x; 1.0093x over previous
import functools

import jax
import jax.numpy as jnp
from jax import lax
from jax.experimental import pallas as pl
from jax.experimental.pallas import tpu as pltpu
from jax.experimental.pallas import tpu_sc as plsc

_NC, _NS = 2, 16
_NW = _NC * _NS
_CH = 256
_TC_BLK = 4096


@functools.lru_cache(maxsize=None)
def _make_transpose(n_rows, d):
    grid = (n_rows + _TC_BLK - 1) // _TC_BLK

    def body(x_ref, o_ref):
        o_ref[...] = x_ref[...].T

    return pl.pallas_call(
        body,
        grid=(grid,),
        in_specs=[pl.BlockSpec((d, _TC_BLK), lambda j: (0, j))],
        out_specs=pl.BlockSpec((_TC_BLK, d), lambda j: (j, 0)),
        out_shape=jax.ShapeDtypeStruct((n_rows, d), jnp.float32),
    )


@functools.lru_cache(maxsize=None)
def _make_gather(t_total, d):
    n_units = t_total // _CH
    upw = n_units // _NW
    assert upw * _NW * _CH == t_total

    mesh = plsc.VectorSubcoreMesh(core_axis_name="c", subcore_axis_name="s")

    @functools.partial(
        pl.kernel,
        mesh=mesh,
        compiler_params=pltpu.CompilerParams(use_tc_tiling_on_sc=False),
        out_type=jax.ShapeDtypeStruct((t_total, d), jnp.float32),
        scratch_types=[
            pltpu.VMEM((2, _CH), jnp.int32),
            pltpu.VMEM((2, _CH, 2 * d), jnp.float32),
            pltpu.SemaphoreType.DMA,
            pltpu.SemaphoreType.DMA,
            pltpu.SemaphoreType.DMA,
            pltpu.SemaphoreType.DMA,
        ],
    )
    def emb(table_hbm, idx_hbm, out_hbm, idx_v, rows_v, g0, g1, w0, w1):
        wid = lax.axis_index("s") * _NC + lax.axis_index("c")
        u0 = wid * upw
        gsem = (g0, g1)
        wsem = (w0, w1)
        gath = [None, None]
        wrb = [None, None]
        for i in range(upw):
            cur = i % 2
            u = u0 + i
            if wrb[cur] is not None:
                wrb[cur].wait()
            pltpu.sync_copy(idx_hbm.at[pl.ds(u * _CH, _CH)],
                            idx_v.at[cur])
            gath[cur] = pltpu.async_copy(
                table_hbm.at[idx_v.at[cur]], rows_v.at[cur], gsem[cur])
            if i > 0:
                prv = 1 - cur
                gath[prv].wait()
                wrb[prv] = pltpu.async_copy(
                    rows_v.at[prv, :, pl.ds(0, d)],
                    out_hbm.at[pl.ds((u - 1) * _CH, _CH)],
                    wsem[prv])
        last = (upw - 1) % 2
        ul = u0 + upw - 1
        gath[last].wait()
        pltpu.sync_copy(rows_v.at[last, :, pl.ds(0, d)],
                        out_hbm.at[pl.ds(ul * _CH, _CH)])
        if wrb[1 - last] is not None:
            wrb[1 - last].wait()

    return emb


def kernel(token_ids, weight):
    b, l = token_ids.shape
    n, d = weight.shape
    idx_flat = token_ids.T.reshape(-1).astype(jnp.int32)
    w_pad = jnp.pad(weight, ((0, 0), (0, d)))
    out = _make_gather(b * l, d)(w_pad, idx_flat)
    return jnp.transpose(out.reshape(l, b, d), (1, 0, 2))
